# Initial kernel scaffold; baseline (speedup 1.0000x reference)
import jax
import jax.numpy as jnp
from jax.experimental import pallas as pl


def kernel(x, edge_index, W1, b1, W2, b2):
    raise NotImplementedError("write your pallas kernel here")


# speedup vs baseline: 9.1402x; 9.1402x over previous
import functools

import jax
import jax.numpy as jnp
from jax import lax
from jax.experimental import pallas as pl
from jax.experimental.pallas import tpu as pltpu
from jax.experimental.pallas import tpu_sc as plsc

N = 10000
E = 160000
IN_DIM = 256
HID_DIM = 128
OUT_DIM = 16

NC, NS = 2, 16
NW = NC * NS
L = 16

CHUNK = 128
EPT = 5120
EPAD = EPT * NW
NPAD = 10240
RPT = NPAD // NS

_mesh = plsc.VectorSubcoreMesh(
    core_axis_name="c", subcore_axis_name="s", num_cores=NC, num_subcores=NS)


@functools.partial(
    pl.kernel,
    out_type=jax.ShapeDtypeStruct((NC, NPAD), jnp.float32),
    mesh=_mesh,
    scratch_types=[
        pltpu.VMEM((CHUNK,), jnp.int32),
        pltpu.VMEM((CHUNK,), jnp.float32),
        pltpu.VMEM_SHARED((NPAD,), jnp.float32),
    ],
)
def _deg_kernel(zeros_hbm, dst_hbm, out_hbm, dstv, ones, acc):
    cid = lax.axis_index("c")
    sid = lax.axis_index("s")
    wid = cid * NS + sid
    ones16 = jnp.ones((L,), jnp.float32)

    @pl.loop(0, CHUNK // L)
    def _fill(j):
        ones[pl.ds(j * L, L)] = ones16

    pltpu.sync_copy(zeros_hbm, acc.at[pl.ds(sid * RPT, RPT)])
    plsc.subcore_barrier()

    @pl.loop(0, EPT // CHUNK)
    def _edges(c):
        off = wid * EPT + c * CHUNK
        pltpu.sync_copy(dst_hbm.at[pl.ds(off, CHUNK)], dstv)
        pltpu.sync_copy(ones, acc.at[dstv], add=True)

    plsc.subcore_barrier()
    pltpu.sync_copy(acc.at[pl.ds(sid * RPT, RPT)],
                    out_hbm.at[cid, pl.ds(sid * RPT, RPT)])


def _make_segsum(D):
    @functools.partial(
        pl.kernel,
        out_type=jax.ShapeDtypeStruct((NC, NPAD, D), jnp.float32),
        mesh=_mesh,
        scratch_types=[
            pltpu.VMEM((CHUNK,), jnp.int32),
            pltpu.VMEM((CHUNK,), jnp.int32),
            pltpu.VMEM((CHUNK, D), jnp.float32),
            pltpu.VMEM_SHARED((NPAD, D), jnp.float32),
        ],
        compiler_params=pltpu.CompilerParams(use_tc_tiling_on_sc=False),
    )
    def _segsum(zeros_hbm, s_hbm, src_hbm, dst_hbm, out_hbm, srcv, dstv, rows, acc):
        cid = lax.axis_index("c")
        sid = lax.axis_index("s")
        wid = cid * NS + sid

        pltpu.sync_copy(zeros_hbm, acc.at[pl.ds(sid * RPT, RPT)])
        plsc.subcore_barrier()

        @pl.loop(0, EPT // CHUNK)
        def _edges(c):
            off = wid * EPT + c * CHUNK
            pltpu.sync_copy(src_hbm.at[pl.ds(off, CHUNK)], srcv)
            pltpu.sync_copy(dst_hbm.at[pl.ds(off, CHUNK)], dstv)
            pltpu.sync_copy(s_hbm.at[srcv], rows)
            pltpu.sync_copy(rows, acc.at[dstv], add=True)

        plsc.subcore_barrier()
        pltpu.sync_copy(acc.at[pl.ds(sid * RPT, RPT)],
                        out_hbm.at[cid, pl.ds(sid * RPT, RPT)])

    return _segsum


_segsum_hid = _make_segsum(HID_DIM)
_segsum_out = _make_segsum(OUT_DIM)


_BLK = 1000


def _tc1_body(x_ref, w1_ref, degp_ref, s1_ref, dis_ref):
    deg = jnp.sum(degp_ref[...], axis=0) + 1.0
    dis = lax.rsqrt(deg)
    xw = jnp.dot(x_ref[...], w1_ref[...], preferred_element_type=jnp.float32)
    s1_ref[...] = xw * dis
    dis_ref[...] = dis


def _tc1(x, W1, degp3):
    return pl.pallas_call(
        _tc1_body,
        grid=(N // _BLK,),
        in_specs=[
            pl.BlockSpec((_BLK, IN_DIM), lambda i: (i, 0)),
            pl.BlockSpec((IN_DIM, HID_DIM), lambda i: (0, 0)),
            pl.BlockSpec((NC, _BLK, 1), lambda i: (0, i, 0)),
        ],
        out_specs=[
            pl.BlockSpec((_BLK, HID_DIM), lambda i: (i, 0)),
            pl.BlockSpec((_BLK, 1), lambda i: (i, 0)),
        ],
        out_shape=[
            jax.ShapeDtypeStruct((N, HID_DIM), jnp.float32),
            jax.ShapeDtypeStruct((N, 1), jnp.float32),
        ],
    )(x, W1, degp3)


def _tc2_body(t1p_ref, s1_ref, dis_ref, b1_ref, w2_ref, s2_ref):
    t = t1p_ref[0] + t1p_ref[1] + s1_ref[...]
    h = jnp.maximum(t * dis_ref[...] + b1_ref[...], 0.0)
    xw2 = jnp.dot(h, w2_ref[...], preferred_element_type=jnp.float32)
    s2_ref[...] = xw2 * dis_ref[...]


def _tc2(t1p, s1, dis, b1r, W2):
    return pl.pallas_call(
        _tc2_body,
        grid=(N // _BLK,),
        in_specs=[
            pl.BlockSpec((NC, _BLK, HID_DIM), lambda i: (0, i, 0)),
            pl.BlockSpec((_BLK, HID_DIM), lambda i: (i, 0)),
            pl.BlockSpec((_BLK, 1), lambda i: (i, 0)),
            pl.BlockSpec((1, HID_DIM), lambda i: (0, 0)),
            pl.BlockSpec((HID_DIM, OUT_DIM), lambda i: (0, 0)),
        ],
        out_specs=pl.BlockSpec((_BLK, OUT_DIM), lambda i: (i, 0)),
        out_shape=jax.ShapeDtypeStruct((N, OUT_DIM), jnp.float32),
    )(t1p, s1, dis, b1r, W2)


def _tc3_body(t2p_ref, s2_ref, dis_ref, b2_ref, o_ref):
    o = (t2p_ref[0] + t2p_ref[1] + s2_ref[...]) * dis_ref[...] + b2_ref[...]
    m = jnp.max(o, axis=1, keepdims=True)
    lse = jnp.log(jnp.sum(jnp.exp(o - m), axis=1, keepdims=True)) + m
    o_ref[...] = o - lse


def _tc3(t2p, s2, dis, b2r):
    return pl.pallas_call(
        _tc3_body,
        grid=(N // _BLK,),
        in_specs=[
            pl.BlockSpec((NC, _BLK, OUT_DIM), lambda i: (0, i, 0)),
            pl.BlockSpec((_BLK, OUT_DIM), lambda i: (i, 0)),
            pl.BlockSpec((_BLK, 1), lambda i: (i, 0)),
            pl.BlockSpec((1, OUT_DIM), lambda i: (0, 0)),
        ],
        out_specs=pl.BlockSpec((_BLK, OUT_DIM), lambda i: (i, 0)),
        out_shape=jax.ShapeDtypeStruct((N, OUT_DIM), jnp.float32),
    )(t2p, s2, dis, b2r)


def kernel(x, edge_index, W1, b1, W2, b2):
    ei = edge_index.astype(jnp.int32)
    pad = EPAD - E
    src = jnp.concatenate([ei[0], jnp.zeros((pad,), jnp.int32)])
    dst = jnp.concatenate([ei[1], jnp.full((pad,), N, jnp.int32)])

    z_deg = jnp.zeros((RPT,), jnp.float32)
    degp = _deg_kernel(z_deg, dst)
    degp3 = degp[:, :N, None]

    s1, dis = _tc1(x, W1, degp3)

    z_hid = jnp.zeros((RPT, HID_DIM), jnp.float32)
    t1p = _segsum_hid(z_hid, s1, src, dst)

    s2 = _tc2(t1p[:, :N], s1, dis, b1.reshape(1, HID_DIM), W2)

    z_out = jnp.zeros((RPT, OUT_DIM), jnp.float32)
    t2p = _segsum_out(z_out, s2, src, dst)

    return _tc3(t2p[:, :N], s2, dis, b2.reshape(1, OUT_DIM))


# speedup vs baseline: 11.7218x; 1.2824x over previous
import functools

import jax
import jax.numpy as jnp
from jax import lax
from jax.experimental import pallas as pl
from jax.experimental.pallas import tpu as pltpu
from jax.experimental.pallas import tpu_sc as plsc

N = 10000
E = 160000
IN_DIM = 256
HID_DIM = 128
OUT_DIM = 16

NC, NS = 2, 16
NW = NC * NS
L = 16

CHUNK = 128
EPT = 5120
EPAD = EPT * NW
NPAD = 10240
RPT = NPAD // NS

_mesh = plsc.VectorSubcoreMesh(
    core_axis_name="c", subcore_axis_name="s", num_cores=NC, num_subcores=NS)


@functools.partial(
    pl.kernel,
    out_type=jax.ShapeDtypeStruct((NC, NPAD), jnp.float32),
    mesh=_mesh,
    scratch_types=[
        pltpu.VMEM((CHUNK,), jnp.int32),
        pltpu.VMEM((CHUNK,), jnp.float32),
        pltpu.VMEM_SHARED((NPAD,), jnp.float32),
    ],
)
def _deg_kernel(zeros_hbm, dst_hbm, out_hbm, dstv, ones, acc):
    cid = lax.axis_index("c")
    sid = lax.axis_index("s")
    wid = cid * NS + sid
    ones16 = jnp.ones((L,), jnp.float32)

    @pl.loop(0, CHUNK // L)
    def _fill(j):
        ones[pl.ds(j * L, L)] = ones16

    pltpu.sync_copy(zeros_hbm, acc.at[pl.ds(sid * RPT, RPT)])
    plsc.subcore_barrier()

    @pl.loop(0, EPT // CHUNK)
    def _edges(c):
        off = wid * EPT + c * CHUNK
        pltpu.sync_copy(dst_hbm.at[pl.ds(off, CHUNK)], dstv)
        pltpu.sync_copy(ones, acc.at[dstv], add=True)

    plsc.subcore_barrier()
    pltpu.sync_copy(acc.at[pl.ds(sid * RPT, RPT)],
                    out_hbm.at[cid, pl.ds(sid * RPT, RPT)])


NCH = EPT // CHUNK


def _make_segsum(D):
    @functools.partial(
        pl.kernel,
        out_type=jax.ShapeDtypeStruct((NC, NPAD, D), jnp.float32),
        mesh=_mesh,
        scratch_types=[
            pltpu.VMEM((NCH, CHUNK), jnp.int32),
            pltpu.VMEM((NCH, CHUNK), jnp.int32),
            pltpu.VMEM((CHUNK, D), jnp.float32),
            pltpu.VMEM((CHUNK, D), jnp.float32),
            pltpu.SemaphoreType.DMA,
            pltpu.SemaphoreType.DMA,
            pltpu.VMEM_SHARED((NPAD, D), jnp.float32),
        ],
        compiler_params=pltpu.CompilerParams(use_tc_tiling_on_sc=False),
    )
    def _segsum(zeros_hbm, s_hbm, src_hbm, dst_hbm, out_hbm,
                srcall, dstall, rows0, rows1, sem0, sem1, acc):
        cid = lax.axis_index("c")
        sid = lax.axis_index("s")
        wid = cid * NS + sid

        pltpu.sync_copy(zeros_hbm, acc.at[pl.ds(sid * RPT, RPT)])
        pltpu.sync_copy(src_hbm.at[pl.ds(wid * NCH, NCH)], srcall)
        pltpu.sync_copy(dst_hbm.at[pl.ds(wid * NCH, NCH)], dstall)
        plsc.subcore_barrier()

        pltpu.async_copy(s_hbm.at[srcall.at[0]], rows0, sem0)

        @pl.loop(0, NCH, step=2)
        def _edges(c):
            pltpu.async_copy(s_hbm.at[srcall.at[c + 1]], rows1, sem1)
            pltpu.make_async_copy(s_hbm.at[srcall.at[c]], rows0, sem0).wait()
            pltpu.sync_copy(rows0, acc.at[dstall.at[c]], add=True)

            @pl.when(c + 2 < NCH)
            def _():
                pltpu.async_copy(s_hbm.at[srcall.at[c + 2]], rows0, sem0)

            pltpu.make_async_copy(s_hbm.at[srcall.at[c + 1]], rows1, sem1).wait()
            pltpu.sync_copy(rows1, acc.at[dstall.at[c + 1]], add=True)

        plsc.subcore_barrier()
        pltpu.sync_copy(acc.at[pl.ds(sid * RPT, RPT)],
                        out_hbm.at[cid, pl.ds(sid * RPT, RPT)])

    return _segsum


_segsum_hid = _make_segsum(HID_DIM)
_segsum_out = _make_segsum(OUT_DIM)


_BLK = 1000


def _tc1_body(x_ref, w1_ref, degp_ref, s1_ref, dis_ref):
    deg = jnp.sum(degp_ref[...], axis=0) + 1.0
    dis = lax.rsqrt(deg)
    xw = jnp.dot(x_ref[...], w1_ref[...], preferred_element_type=jnp.float32)
    s1_ref[...] = xw * dis
    dis_ref[...] = dis


def _tc1(x, W1, degp3):
    return pl.pallas_call(
        _tc1_body,
        grid=(N // _BLK,),
        in_specs=[
            pl.BlockSpec((_BLK, IN_DIM), lambda i: (i, 0)),
            pl.BlockSpec((IN_DIM, HID_DIM), lambda i: (0, 0)),
            pl.BlockSpec((NC, _BLK, 1), lambda i: (0, i, 0)),
        ],
        out_specs=[
            pl.BlockSpec((_BLK, HID_DIM), lambda i: (i, 0)),
            pl.BlockSpec((_BLK, 1), lambda i: (i, 0)),
        ],
        out_shape=[
            jax.ShapeDtypeStruct((N, HID_DIM), jnp.float32),
            jax.ShapeDtypeStruct((N, 1), jnp.float32),
        ],
    )(x, W1, degp3)


def _tc2_body(t1p_ref, s1_ref, dis_ref, b1_ref, w2_ref, s2_ref):
    t = t1p_ref[0] + t1p_ref[1] + s1_ref[...]
    h = jnp.maximum(t * dis_ref[...] + b1_ref[...], 0.0)
    xw2 = jnp.dot(h, w2_ref[...], preferred_element_type=jnp.float32)
    s2_ref[...] = xw2 * dis_ref[...]


def _tc2(t1p, s1, dis, b1r, W2):
    return pl.pallas_call(
        _tc2_body,
        grid=(N // _BLK,),
        in_specs=[
            pl.BlockSpec((NC, _BLK, HID_DIM), lambda i: (0, i, 0)),
            pl.BlockSpec((_BLK, HID_DIM), lambda i: (i, 0)),
            pl.BlockSpec((_BLK, 1), lambda i: (i, 0)),
            pl.BlockSpec((1, HID_DIM), lambda i: (0, 0)),
            pl.BlockSpec((HID_DIM, OUT_DIM), lambda i: (0, 0)),
        ],
        out_specs=pl.BlockSpec((_BLK, OUT_DIM), lambda i: (i, 0)),
        out_shape=jax.ShapeDtypeStruct((N, OUT_DIM), jnp.float32),
    )(t1p, s1, dis, b1r, W2)


def _tc3_body(t2p_ref, s2_ref, dis_ref, b2_ref, o_ref):
    o = (t2p_ref[0] + t2p_ref[1] + s2_ref[...]) * dis_ref[...] + b2_ref[...]
    m = jnp.max(o, axis=1, keepdims=True)
    lse = jnp.log(jnp.sum(jnp.exp(o - m), axis=1, keepdims=True)) + m
    o_ref[...] = o - lse


def _tc3(t2p, s2, dis, b2r):
    return pl.pallas_call(
        _tc3_body,
        grid=(N // _BLK,),
        in_specs=[
            pl.BlockSpec((NC, _BLK, OUT_DIM), lambda i: (0, i, 0)),
            pl.BlockSpec((_BLK, OUT_DIM), lambda i: (i, 0)),
            pl.BlockSpec((_BLK, 1), lambda i: (i, 0)),
            pl.BlockSpec((1, OUT_DIM), lambda i: (0, 0)),
        ],
        out_specs=pl.BlockSpec((_BLK, OUT_DIM), lambda i: (i, 0)),
        out_shape=jax.ShapeDtypeStruct((N, OUT_DIM), jnp.float32),
    )(t2p, s2, dis, b2r)


def kernel(x, edge_index, W1, b1, W2, b2):
    ei = edge_index.astype(jnp.int32)
    pad = EPAD - E
    dum = N + jnp.arange(pad, dtype=jnp.int32) % (NPAD - N)
    src = jnp.concatenate([ei[0], jnp.zeros((pad,), jnp.int32)])
    dst = jnp.concatenate([ei[1], dum])
    src2d = src.reshape(EPAD // CHUNK, CHUNK)
    dst2d = dst.reshape(EPAD // CHUNK, CHUNK)

    z_deg = jnp.zeros((RPT,), jnp.float32)
    degp = _deg_kernel(z_deg, dst)
    degp3 = degp[:, :N, None]

    s1, dis = _tc1(x, W1, degp3)

    z_hid = jnp.zeros((RPT, HID_DIM), jnp.float32)
    t1p = _segsum_hid(z_hid, s1, src2d, dst2d)

    s2 = _tc2(t1p[:, :N], s1, dis, b1.reshape(1, HID_DIM), W2)

    z_out = jnp.zeros((RPT, OUT_DIM), jnp.float32)
    t2p = _segsum_out(z_out, s2, src2d, dst2d)

    return _tc3(t2p[:, :N], s2, dis, b2.reshape(1, OUT_DIM))
